# baseline (speedup 1.0000x reference)
import functools

import jax
import jax.numpy as jnp
from jax import lax
from jax.experimental import pallas as pl
from jax.experimental.pallas import tpu as pltpu
from jax.experimental.pallas import tpu_sc as plsc

_D = 128
_LANES = 16
_CH = 128


def _pack_bf16_words(even, odd):
    ie = jax.lax.bitcast_convert_type(
        even.astype(jnp.bfloat16).astype(jnp.float32), jnp.uint32)
    io = jax.lax.bitcast_convert_type(
        odd.astype(jnp.bfloat16).astype(jnp.float32), jnp.uint32)
    word = jnp.bitwise_or(jnp.right_shift(ie, 16),
                          jnp.bitwise_and(io, jnp.uint32(0xFFFF0000)))
    return jax.lax.bitcast_convert_type(word, jnp.int32)


def _tables_body(emb_ref, wae_ref, wao_ref, wbe_ref, wbo_ref,
                 be_ref, bo_ref, ta_ref, tb_ref):
    emb = emb_ref[...]
    ae = jnp.dot(emb, wae_ref[...], preferred_element_type=jnp.float32) + be_ref[...]
    ao = jnp.dot(emb, wao_ref[...], preferred_element_type=jnp.float32) + bo_ref[...]
    ta_ref[...] = _pack_bf16_words(ae, ao)
    tb_ref[...] = _pack_bf16_words(
        jnp.dot(emb, wbe_ref[...], preferred_element_type=jnp.float32),
        jnp.dot(emb, wbo_ref[...], preferred_element_type=jnp.float32))


def _node_tables(emb, wae, wao, wbe, wbo, be, bo):
    n = emb.shape[0]
    return pl.pallas_call(
        _tables_body,
        out_shape=(
            jax.ShapeDtypeStruct((n, _D // 2), jnp.int32),
            jax.ShapeDtypeStruct((n, _D // 2), jnp.int32),
        ),
    )(emb, wae, wao, wbe, wbo, be, bo)


_K_SLOW = 80


@functools.cache
def _make_sc_scores(e_pad: int):
    info = plsc.get_sparse_core_info()
    nc, ns = info.num_cores, info.num_subcores
    nw = nc * ns
    per_w = e_pad // nw
    n_chunks = per_w // _CH
    k0 = _K_SLOW
    k1 = 2 * n_chunks - k0
    kmax = max(k0, k1)
    mesh = plsc.VectorSubcoreMesh(core_axis_name="c", subcore_axis_name="s")

    @functools.partial(
        pl.kernel,
        mesh=mesh,
        compiler_params=pltpu.CompilerParams(
            needs_layout_passes=False, use_tc_tiling_on_sc=False),
        out_type=jax.ShapeDtypeStruct((e_pad,), jnp.float32),
        scratch_types=[
            pltpu.VMEM((kmax * _CH,), jnp.int32),
            pltpu.VMEM((_CH,), jnp.int32),
            pltpu.VMEM((_CH,), jnp.int32),
            pltpu.VMEM((_CH,), jnp.int32),
            pltpu.VMEM((_CH,), jnp.int32),
            pltpu.VMEM((_CH, _D // 2), jnp.int32),
            pltpu.VMEM((_CH, _D // 2), jnp.int32),
            pltpu.VMEM((_CH, _D // 2), jnp.int32),
            pltpu.VMEM((_CH, _D // 2), jnp.int32),
            pltpu.VMEM((kmax * _CH,), jnp.float32),
            pltpu.VMEM((_D,), jnp.float32),
            pltpu.VMEM((_LANES, _LANES), jnp.float32),
            pltpu.VMEM_SHARED((10000, _D // 2), jnp.int32),
            pltpu.SemaphoreType.DMA,
            pltpu.SemaphoreType.DMA,
            pltpu.SemaphoreType.DMA,
            pltpu.SemaphoreType.DMA,
        ],
    )
    def sc_scores(ta_hbm, tb_hbm, pk_hbm, w2_hbm, out_hbm,
                  pk_v, src0_v, dst0_v, src1_v, dst1_v,
                  a0_v, b0_v, a1_v, b1_v, s_all, w2_v, m_v,
                  ta_sh, sa0, sb0, sa1, sb1):
        sid = lax.axis_index("s")
        cid = lax.axis_index("c")
        base = pl.multiple_of(
            sid * (k0 + k1) * _CH + cid * k0 * _CH, _CH)
        my_pairs = jnp.where(cid == 0, k0 // 2, k1 // 2)
        slab = 10000 // ns
        row0 = sid * slab
        pltpu.sync_copy(ta_hbm.at[pl.ds(row0, slab)],
                        ta_sh.at[pl.ds(row0, slab)])
        plsc.subcore_barrier()
        pltpu.sync_copy(w2_hbm, w2_v)

        @pl.when(cid == 0)
        def _():
            pltpu.sync_copy(pk_hbm.at[pl.ds(base, k0 * _CH)],
                            pk_v.at[pl.ds(0, k0 * _CH)])

        @pl.when(cid == 1)
        def _():
            pltpu.sync_copy(pk_hbm.at[pl.ds(base, k1 * _CH)],
                            pk_v.at[pl.ds(0, k1 * _CH)])
        w2regs = [w2_v[pl.ds(_LANES * c, _LANES)] for c in range(_D // _LANES)]
        lanes = lax.iota(jnp.int32, 16)
        slots = ((src0_v, dst0_v, a0_v, b0_v, sa0, sb0),
                 (src1_v, dst1_v, a1_v, b1_v, sa1, sb1))

        def issue(j, slot):
            src_v, dst_v, a_v, b_v, sa, sb = slots[slot]
            off = pl.multiple_of(j * _CH, _CH)
            for k in range(_CH // _LANES):
                pk = pk_v[pl.ds(off + _LANES * k, _LANES)]
                src_v[pl.ds(_LANES * k, _LANES)] = jnp.bitwise_and(pk, 0xFFFF)
                dst_v[pl.ds(_LANES * k, _LANES)] = jnp.right_shift(pk, 16)
            pltpu.make_async_copy(ta_sh.at[src_v], a_v, sa).start()
            pltpu.make_async_copy(tb_hbm.at[dst_v], b_v, sb).start()

        def wait(slot):
            src_v, dst_v, a_v, b_v, sa, sb = slots[slot]
            pltpu.make_async_copy(ta_sh.at[src_v], a_v, sa).wait()
            pltpu.make_async_copy(tb_hbm.at[dst_v], b_v, sb).wait()

        def compute(j, slot):
            _, _, a_v, b_v, _, _ = slots[slot]
            s_off = pl.multiple_of(j * _CH, _CH)

            zero = jnp.zeros((2 * _LANES,), jnp.bfloat16)
            n_p = _D // (2 * _LANES)

            def relu_pair(i, p):
                a = plsc.bitcast(a_v[i, pl.ds(_LANES * p, _LANES)], jnp.bfloat16)
                b = plsc.bitcast(b_v[i, pl.ds(_LANES * p, _LANES)], jnp.bfloat16)
                return plsc.unpack(jnp.maximum(a + b, zero),
                                   format=plsc.PackFormat.INTERLEAVED)

            def grp_body(g, carry2):
                for e in range(0, _LANES, 2):
                    i0 = g * _LANES + e
                    i1 = i0 + 1
                    lo0, hi0 = relu_pair(i0, 0)
                    lo1, hi1 = relu_pair(i1, 0)
                    x0 = lo0 * w2regs[0]
                    y0 = hi0 * w2regs[1]
                    x1 = lo1 * w2regs[0]
                    y1 = hi1 * w2regs[1]
                    for p in range(1, n_p):
                        lo0, hi0 = relu_pair(i0, p)
                        lo1, hi1 = relu_pair(i1, p)
                        x0 = x0 + lo0 * w2regs[2 * p]
                        y0 = y0 + hi0 * w2regs[2 * p + 1]
                        x1 = x1 + lo1 * w2regs[2 * p]
                        y1 = y1 + hi1 * w2regs[2 * p + 1]
                    m_v[e, :] = x0 + y0
                    m_v[e + 1, :] = x1 + y1
                tot = jnp.zeros((_LANES,), jnp.float32)
                for c in range(_LANES):
                    col = jnp.full((_LANES,), c, jnp.int32)
                    tot = tot + plsc.load_gather(m_v, [lanes, col])
                s_all[pl.ds(s_off + g * _LANES, _LANES)] = tot
                return carry2

            lax.fori_loop(0, _CH // _LANES, grp_body, 0)

        issue(0, 0)

        def pair_body(jj, carry):
            j0 = jj * 2
            issue(j0 + 1, 1)
            wait(0)
            compute(j0, 0)
            pl.when(jj + 1 < my_pairs)(lambda: issue(j0 + 2, 0))
            wait(1)
            compute(j0 + 1, 1)
            return carry

        lax.fori_loop(0, my_pairs, pair_body, 0)

        @pl.when(cid == 0)
        def _():
            pltpu.sync_copy(s_all.at[pl.ds(0, k0 * _CH)],
                            out_hbm.at[pl.ds(base, k0 * _CH)])

        @pl.when(cid == 1)
        def _():
            pltpu.sync_copy(s_all.at[pl.ds(0, k1 * _CH)],
                            out_hbm.at[pl.ds(base, k1 * _CH)])

    return sc_scores


def _softmax_body(x_ref, o_ref):
    x = x_ref[...]
    m = jnp.max(x)
    e = jnp.exp(x - m)
    o_ref[...] = e / jnp.sum(e)


def _softmax(x2d):
    return pl.pallas_call(
        _softmax_body,
        out_shape=jax.ShapeDtypeStruct(x2d.shape, jnp.float32),
    )(x2d)


def kernel(edge_list, node_embeddings, W1, b1, W2, b2):
    n_edges = edge_list.shape[0]
    d = node_embeddings.shape[1]
    src = edge_list[:, 0].astype(jnp.int32)
    dst = edge_list[:, 1].astype(jnp.int32)
    packed = jnp.bitwise_or(src, jnp.left_shift(dst, 16))

    stride = 32 * 2 * _CH
    e_pad = ((n_edges + stride - 1) // stride) * stride
    pad = e_pad - n_edges
    if pad:
        packed = jnp.concatenate([packed, jnp.zeros((pad,), jnp.int32)])

    w1a_t = W1[:, :d].T
    w1b_t = W1[:, d:].T
    ta_w, tb_w = _node_tables(
        node_embeddings,
        w1a_t[:, 0::2], w1a_t[:, 1::2], w1b_t[:, 0::2], w1b_t[:, 1::2],
        b1[0::2].reshape(1, d // 2), b1[1::2].reshape(1, d // 2))

    w2_arr = W2.reshape(d // 32, 16, 2).transpose(0, 2, 1).reshape(d)
    scores = _make_sc_scores(e_pad)(ta_w, tb_w, packed, w2_arr)
    scores = scores[:n_edges]
    probs = _softmax(scores.reshape(n_edges // _D, _D)).reshape(n_edges)
    return probs


# speedup vs baseline: 1.1376x; 1.1376x over previous
import functools

import jax
import jax.numpy as jnp
from jax import lax
from jax.experimental import pallas as pl
from jax.experimental.pallas import tpu as pltpu
from jax.experimental.pallas import tpu_sc as plsc

_D = 128
_LANES = 16
_CH = 128


def _pack_bf16_words(even, odd):
    ie = jax.lax.bitcast_convert_type(
        even.astype(jnp.bfloat16).astype(jnp.float32), jnp.uint32)
    io = jax.lax.bitcast_convert_type(
        odd.astype(jnp.bfloat16).astype(jnp.float32), jnp.uint32)
    word = jnp.bitwise_or(jnp.right_shift(ie, 16),
                          jnp.bitwise_and(io, jnp.uint32(0xFFFF0000)))
    return jax.lax.bitcast_convert_type(word, jnp.int32)


def _tables_body(emb_ref, wae_ref, wao_ref, wbe_ref, wbo_ref,
                 be_ref, bo_ref, ta_ref, tb_ref):
    emb = emb_ref[...]
    ae = jnp.dot(emb, wae_ref[...], preferred_element_type=jnp.float32) + be_ref[...]
    ao = jnp.dot(emb, wao_ref[...], preferred_element_type=jnp.float32) + bo_ref[...]
    ta_ref[...] = _pack_bf16_words(ae, ao)
    tb_ref[...] = _pack_bf16_words(
        jnp.dot(emb, wbe_ref[...], preferred_element_type=jnp.float32),
        jnp.dot(emb, wbo_ref[...], preferred_element_type=jnp.float32))


def _node_tables(emb, wae, wao, wbe, wbo, be, bo):
    n = emb.shape[0]
    return pl.pallas_call(
        _tables_body,
        out_shape=(
            jax.ShapeDtypeStruct((n, _D // 2), jnp.int32),
            jax.ShapeDtypeStruct((n, _D // 2), jnp.int32),
        ),
    )(emb, wae, wao, wbe, wbo, be, bo)


_K_SLOW = 110


@functools.cache
def _make_sc_scores(e_pad: int):
    info = plsc.get_sparse_core_info()
    nc, ns = info.num_cores, info.num_subcores
    nw = nc * ns
    per_w = e_pad // nw
    n_chunks = per_w // _CH
    k0 = _K_SLOW
    k1 = 2 * n_chunks - k0
    kmax = max(k0, k1)
    mesh = plsc.VectorSubcoreMesh(core_axis_name="c", subcore_axis_name="s")

    @functools.partial(
        pl.kernel,
        mesh=mesh,
        compiler_params=pltpu.CompilerParams(
            needs_layout_passes=False, use_tc_tiling_on_sc=False),
        out_type=jax.ShapeDtypeStruct((e_pad,), jnp.float32),
        scratch_types=[
            pltpu.VMEM((kmax * _CH,), jnp.int32),
            pltpu.VMEM((_CH,), jnp.int32),
            pltpu.VMEM((_CH,), jnp.int32),
            pltpu.VMEM((_CH,), jnp.int32),
            pltpu.VMEM((_CH,), jnp.int32),
            pltpu.VMEM((_CH, _D // 2), jnp.int32),
            pltpu.VMEM((_CH, _D // 2), jnp.int32),
            pltpu.VMEM((_CH, _D // 2), jnp.int32),
            pltpu.VMEM((_CH, _D // 2), jnp.int32),
            pltpu.VMEM((kmax * _CH,), jnp.float32),
            pltpu.VMEM((_D,), jnp.float32),
            pltpu.VMEM((_LANES, _LANES), jnp.float32),
            pltpu.VMEM_SHARED((10000, _D // 2), jnp.int32),
            pltpu.SemaphoreType.DMA,
            pltpu.SemaphoreType.DMA,
            pltpu.SemaphoreType.DMA,
            pltpu.SemaphoreType.DMA,
        ],
    )
    def sc_scores(ta_hbm, tb_hbm, pk_hbm, w2_hbm, out_hbm,
                  pk_v, src0_v, dst0_v, src1_v, dst1_v,
                  a0_v, b0_v, a1_v, b1_v, s_all, w2_v, m_v,
                  ta_sh, sa0, sb0, sa1, sb1):
        sid = lax.axis_index("s")
        cid = lax.axis_index("c")
        base = pl.multiple_of(
            sid * (k0 + k1) * _CH + cid * k0 * _CH, _CH)
        my_pairs = jnp.where(cid == 0, k0 // 2, k1 // 2)
        slab = 10000 // ns
        row0 = sid * slab
        pltpu.sync_copy(ta_hbm.at[pl.ds(row0, slab)],
                        ta_sh.at[pl.ds(row0, slab)])
        plsc.subcore_barrier()
        pltpu.sync_copy(w2_hbm, w2_v)

        @pl.when(cid == 0)
        def _():
            pltpu.sync_copy(pk_hbm.at[pl.ds(base, k0 * _CH)],
                            pk_v.at[pl.ds(0, k0 * _CH)])

        @pl.when(cid == 1)
        def _():
            pltpu.sync_copy(pk_hbm.at[pl.ds(base, k1 * _CH)],
                            pk_v.at[pl.ds(0, k1 * _CH)])
        w2regs = [w2_v[pl.ds(_LANES * c, _LANES)] for c in range(_D // _LANES)]
        lanes = lax.iota(jnp.int32, 16)
        slots = ((src0_v, dst0_v, a0_v, b0_v, sa0, sb0),
                 (src1_v, dst1_v, a1_v, b1_v, sa1, sb1))

        def issue(j, slot):
            src_v, dst_v, a_v, b_v, sa, sb = slots[slot]
            off = pl.multiple_of(j * _CH, _CH)
            for k in range(_CH // _LANES):
                pk = pk_v[pl.ds(off + _LANES * k, _LANES)]
                src_v[pl.ds(_LANES * k, _LANES)] = jnp.bitwise_and(pk, 0xFFFF)
                dst_v[pl.ds(_LANES * k, _LANES)] = jnp.right_shift(pk, 16)
            pltpu.make_async_copy(ta_sh.at[src_v], a_v, sa).start()
            pltpu.make_async_copy(tb_hbm.at[dst_v], b_v, sb).start()

        def wait(slot):
            src_v, dst_v, a_v, b_v, sa, sb = slots[slot]
            pltpu.make_async_copy(ta_sh.at[src_v], a_v, sa).wait()
            pltpu.make_async_copy(tb_hbm.at[dst_v], b_v, sb).wait()

        def compute(j, slot):
            _, _, a_v, b_v, _, _ = slots[slot]
            s_off = pl.multiple_of(j * _CH, _CH)

            zero = jnp.zeros((2 * _LANES,), jnp.bfloat16)
            n_p = _D // (2 * _LANES)

            def relu_pair(i, p):
                a = plsc.bitcast(a_v[i, pl.ds(_LANES * p, _LANES)], jnp.bfloat16)
                b = plsc.bitcast(b_v[i, pl.ds(_LANES * p, _LANES)], jnp.bfloat16)
                return plsc.unpack(jnp.maximum(a + b, zero),
                                   format=plsc.PackFormat.INTERLEAVED)

            def grp_body(g, carry2):
                for e in range(0, _LANES, 2):
                    i0 = g * _LANES + e
                    i1 = i0 + 1
                    lo0, hi0 = relu_pair(i0, 0)
                    lo1, hi1 = relu_pair(i1, 0)
                    x0 = lo0 * w2regs[0]
                    y0 = hi0 * w2regs[1]
                    x1 = lo1 * w2regs[0]
                    y1 = hi1 * w2regs[1]
                    for p in range(1, n_p):
                        lo0, hi0 = relu_pair(i0, p)
                        lo1, hi1 = relu_pair(i1, p)
                        x0 = x0 + lo0 * w2regs[2 * p]
                        y0 = y0 + hi0 * w2regs[2 * p + 1]
                        x1 = x1 + lo1 * w2regs[2 * p]
                        y1 = y1 + hi1 * w2regs[2 * p + 1]
                    m_v[e, :] = x0 + y0
                    m_v[e + 1, :] = x1 + y1
                tot = jnp.zeros((_LANES,), jnp.float32)
                for c in range(_LANES):
                    col = jnp.full((_LANES,), c, jnp.int32)
                    tot = tot + plsc.load_gather(m_v, [lanes, col])
                s_all[pl.ds(s_off + g * _LANES, _LANES)] = tot
                return carry2

            lax.fori_loop(0, _CH // _LANES, grp_body, 0)

        issue(0, 0)

        def pair_body(jj, carry):
            j0 = jj * 2
            issue(j0 + 1, 1)
            wait(0)
            compute(j0, 0)
            pl.when(jj + 1 < my_pairs)(lambda: issue(j0 + 2, 0))
            wait(1)
            compute(j0 + 1, 1)
            return carry

        lax.fori_loop(0, my_pairs, pair_body, 0)

        @pl.when(cid == 0)
        def _():
            pltpu.sync_copy(s_all.at[pl.ds(0, k0 * _CH)],
                            out_hbm.at[pl.ds(base, k0 * _CH)])

        @pl.when(cid == 1)
        def _():
            pltpu.sync_copy(s_all.at[pl.ds(0, k1 * _CH)],
                            out_hbm.at[pl.ds(base, k1 * _CH)])

    return sc_scores


def _softmax_body(x_ref, o_ref):
    x = x_ref[...]
    m = jnp.max(x)
    e = jnp.exp(x - m)
    o_ref[...] = e / jnp.sum(e)


def _softmax(x2d):
    return pl.pallas_call(
        _softmax_body,
        out_shape=jax.ShapeDtypeStruct(x2d.shape, jnp.float32),
    )(x2d)


def kernel(edge_list, node_embeddings, W1, b1, W2, b2):
    n_edges = edge_list.shape[0]
    d = node_embeddings.shape[1]
    src = edge_list[:, 0].astype(jnp.int32)
    dst = edge_list[:, 1].astype(jnp.int32)
    packed = jnp.bitwise_or(src, jnp.left_shift(dst, 16))

    stride = 32 * 2 * _CH
    e_pad = ((n_edges + stride - 1) // stride) * stride
    pad = e_pad - n_edges
    if pad:
        packed = jnp.concatenate([packed, jnp.zeros((pad,), jnp.int32)])

    w1a_t = W1[:, :d].T
    w1b_t = W1[:, d:].T
    ta_w, tb_w = _node_tables(
        node_embeddings,
        w1a_t[:, 0::2], w1a_t[:, 1::2], w1b_t[:, 0::2], w1b_t[:, 1::2],
        b1[0::2].reshape(1, d // 2), b1[1::2].reshape(1, d // 2))

    w2_arr = W2.reshape(d // 32, 16, 2).transpose(0, 2, 1).reshape(d)
    scores = _make_sc_scores(e_pad)(ta_w, tb_w, packed, w2_arr)
    scores = scores[:n_edges]
    probs = _softmax(scores.reshape(n_edges // _D, _D)).reshape(n_edges)
    return probs
